# Initial kernel scaffold; baseline (speedup 1.0000x reference)
import jax
import jax.numpy as jnp
from jax.experimental import pallas as pl


def kernel(adj_t, edges, embeds, field):
    raise NotImplementedError("write your pallas kernel here")


# speedup vs baseline: 28.5273x; 28.5273x over previous
import functools

import jax
import jax.numpy as jnp
from jax import lax
from jax.experimental import pallas as pl

N_HEADS = 4
N_NODES = 10000
EMB_DIM = 128
N_BATCH = 512
N_NEAREST = 8
N_SENTINELS = 8

QB = 128
NQ = 2 * N_BATCH


def _topk_body(q_ref, e_ref, idx_ref):
    q = q_ref[0]
    e = e_ref[0]
    qn = jnp.sum(q * q, axis=1, keepdims=True)
    kn = jnp.sum(e * e, axis=1)
    prod = lax.dot_general(q, e, (((1,), (1,)), ((), ())),
                           preferred_element_type=jnp.float32)
    d2 = jnp.maximum(qn + kn[None, :] - 2.0 * prod, 0.0)
    n = d2.shape[1]
    iota = lax.broadcasted_iota(jnp.int32, d2.shape, 1)
    cols = []
    for _ in range(N_NEAREST + 1):
        m = jnp.min(d2, axis=1, keepdims=True)
        am = jnp.min(jnp.where(d2 == m, iota, n), axis=1)
        cols.append(am)
        d2 = jnp.where(iota == am[:, None], jnp.inf, d2)
    idx_ref[0] = jnp.stack(cols, axis=1)


def _topk(qrows, embeds):
    grid = (N_HEADS, NQ // QB)
    return pl.pallas_call(
        _topk_body,
        grid=grid,
        in_specs=[
            pl.BlockSpec((1, QB, EMB_DIM), lambda h, b: (h, b, 0)),
            pl.BlockSpec((1, N_NODES, EMB_DIM), lambda h, b: (h, 0, 0)),
        ],
        out_specs=pl.BlockSpec((1, QB, N_NEAREST + 1), lambda h, b: (h, b, 0)),
        out_shape=jax.ShapeDtypeStruct((N_HEADS, NQ, N_NEAREST + 1), jnp.int32),
    )(qrows, embeds)


def _dots_body(q_ref, f_ref, s_ref, d2_ref, lg_ref):
    q = q_ref[...]
    f = f_ref[...]
    s = s_ref[...]
    diff = q[:, None, :] - s
    d2_ref[...] = jnp.sum(diff * diff, axis=2)
    lg_ref[...] = jnp.sum(diff * f[:, None, :], axis=2)


def _dots(qrows, frows, srows):
    R = N_HEADS * NQ
    RB = 512
    grid = (R // RB,)
    return pl.pallas_call(
        _dots_body,
        grid=grid,
        in_specs=[
            pl.BlockSpec((RB, EMB_DIM), lambda i: (i, 0)),
            pl.BlockSpec((RB, EMB_DIM), lambda i: (i, 0)),
            pl.BlockSpec((RB, N_NEAREST, EMB_DIM), lambda i: (i, 0, 0)),
        ],
        out_specs=[
            pl.BlockSpec((RB, N_NEAREST), lambda i: (i, 0)),
            pl.BlockSpec((RB, N_NEAREST), lambda i: (i, 0)),
        ],
        out_shape=[
            jax.ShapeDtypeStruct((R, N_NEAREST), jnp.float32),
            jax.ShapeDtypeStruct((R, N_NEAREST), jnp.float32),
        ],
    )(qrows, frows, srows)


def _combine_body(d2_ref, lg_ref, out_ref):
    d2 = d2_ref[...]
    lg = lg_ref[...]
    dist = jnp.sqrt(d2)
    e = jnp.exp(1.0 - dist)
    num = jnp.sum(e * lg, axis=2)
    den = jnp.sum(e, axis=2)
    num_t = num[:, :N_BATCH] + num[:, N_BATCH:]
    den_t = den[:, :N_BATCH] + den[:, N_BATCH:] + float(N_SENTINELS)
    softmin = num_t / den_t
    preds = jnp.mean(softmin, axis=0)
    out_ref[...] = (1.0 / (1.0 + jnp.exp(-preds)))[None, :]


def _combine(d2, lg):
    return pl.pallas_call(
        _combine_body,
        out_shape=jax.ShapeDtypeStruct((1, N_BATCH), jnp.float32),
    )(d2, lg)


@jax.jit
def kernel(adj_t, edges, embeds, field):
    nodes = jnp.concatenate([edges[0], edges[1]]).astype(jnp.int32)
    qrows = jnp.take(embeds, nodes, axis=1)
    frows = jnp.take(field, nodes, axis=1)

    idx9 = _topk(qrows, embeds)
    samples = idx9[:, :, 1:]
    srows = embeds[jnp.arange(N_HEADS)[:, None, None], samples]

    d2, lg = _dots(qrows.reshape(-1, EMB_DIM), frows.reshape(-1, EMB_DIM),
                   srows.reshape(-1, N_NEAREST, EMB_DIM))
    d2 = d2.reshape(N_HEADS, NQ, N_NEAREST)
    lg = lg.reshape(N_HEADS, NQ, N_NEAREST)
    out = _combine(d2, lg)
    return out.reshape(N_BATCH)
